# Initial kernel scaffold; baseline (speedup 1.0000x reference)
import jax
import jax.numpy as jnp
from jax.experimental import pallas as pl


def kernel(vertices, faces):
    raise NotImplementedError("write your pallas kernel here")


# speedup vs baseline: 10.2952x; 10.2952x over previous
import functools

import jax
import jax.numpy as jnp
from jax import lax
from jax.experimental import pallas as pl
from jax.experimental.pallas import tpu as pltpu
from jax.experimental.pallas import tpu_sc as plsc

B = 32
V = 25000
F = 50000
L = 16
NC, NS = 2, 16

CH = 2000
NCHUNK = F // CH
FE = 3 * CH
OUTC = 9 * CH
NJ = FE // L


def _body(verts_hbm, faces_hbm, out_hbm, verts_v, faces_v, out_v):
    wid = lax.axis_index("s") * NC + lax.axis_index("c")
    pltpu.sync_copy(verts_hbm.at[pl.ds(wid * (V * 3), V * 3)], verts_v)

    iota = lax.iota(jnp.int32, L)
    div3, rem3 = [], []
    for p in range(3):
        base = 16 * p + iota
        q = (base * 21846) >> 16
        div3.append(q)
        rem3.append(base - 3 * q)

    def chunk_work(c):
        pltpu.sync_copy(
            faces_hbm.at[pl.ds(wid * (F * 3) + c * FE, FE)], faces_v
        )

        def step(j, carry):
            fb = jnp.full((L,), j * L, jnp.int32)
            for p in range(3):
                fidx = div3[p] + fb
                g = plsc.load_gather(faces_v, [fidx])
                elem = g * 3 + rem3[p]
                vals = plsc.load_gather(verts_v, [elem])
                out_v[pl.ds(j * 48 + p * L, L)] = vals
            return carry

        lax.fori_loop(0, NJ, step, 0)
        pltpu.sync_copy(
            out_v, out_hbm.at[pl.ds(wid * (F * 9) + c * OUTC, OUTC)]
        )

    for c in range(NCHUNK):
        chunk_work(c)


@functools.partial(jax.jit, static_argnames=())
def kernel(vertices, faces):
    verts_flat = vertices.reshape(B * V * 3)
    faces_flat = faces.reshape(B * F * 3)
    mesh = plsc.VectorSubcoreMesh(
        core_axis_name="c", subcore_axis_name="s", num_cores=NC, num_subcores=NS
    )
    out = pl.kernel(
        _body,
        out_type=jax.ShapeDtypeStruct((B * F * 9,), jnp.float32),
        mesh=mesh,
        compiler_params=pltpu.CompilerParams(needs_layout_passes=False),
        scratch_types=[
            pltpu.VMEM((V * 3,), jnp.float32),
            pltpu.VMEM((FE,), jnp.int32),
            pltpu.VMEM((OUTC,), jnp.float32),
        ],
    )(verts_flat, faces_flat)
    return out.reshape(B, F, 9)


# speedup vs baseline: 10.6459x; 1.0341x over previous
import functools

import jax
import jax.numpy as jnp
from jax import lax
from jax.experimental import pallas as pl
from jax.experimental.pallas import tpu as pltpu
from jax.experimental.pallas import tpu_sc as plsc

B = 32
V = 25000
F = 50000
L = 16
NC, NS = 2, 16

CH = 2000
NCHUNK = F // CH
FE = 3 * CH
OUTC = 9 * CH
NJ = FE // L


def _body(verts_hbm, faces_hbm, out_hbm, verts_v, faces_v, out_v):
    wid = lax.axis_index("s") * NC + lax.axis_index("c")
    pltpu.sync_copy(verts_hbm.at[pl.ds(wid * (V * 3), V * 3)], verts_v)

    iota = lax.iota(jnp.int32, L)
    div3, rem3 = [], []
    for p in range(3):
        base = 16 * p + iota
        q = (base * 21846) >> 16
        div3.append(q)
        rem3.append(base - 3 * q)

    def chunk_work(c):
        pltpu.sync_copy(
            faces_hbm.at[pl.ds(wid * (F * 3) + c * FE, FE)], faces_v
        )

        @plsc.parallel_loop(0, NJ, step=1, unroll=4)
        def _loop(j):
            fb = jnp.full((L,), j * L, jnp.int32)
            for p in range(3):
                fidx = div3[p] + fb
                g = plsc.load_gather(faces_v, [fidx])
                elem = g * 3 + rem3[p]
                vals = plsc.load_gather(verts_v, [elem])
                out_v[pl.ds(j * 48 + p * L, L)] = vals
        pltpu.sync_copy(
            out_v, out_hbm.at[pl.ds(wid * (F * 9) + c * OUTC, OUTC)]
        )

    for c in range(NCHUNK):
        chunk_work(c)


@functools.partial(jax.jit, static_argnames=())
def kernel(vertices, faces):
    verts_flat = vertices.reshape(B * V * 3)
    faces_flat = faces.reshape(B * F * 3)
    mesh = plsc.VectorSubcoreMesh(
        core_axis_name="c", subcore_axis_name="s", num_cores=NC, num_subcores=NS
    )
    out = pl.kernel(
        _body,
        out_type=jax.ShapeDtypeStruct((B * F * 9,), jnp.float32),
        mesh=mesh,
        compiler_params=pltpu.CompilerParams(needs_layout_passes=False),
        scratch_types=[
            pltpu.VMEM((V * 3,), jnp.float32),
            pltpu.VMEM((FE,), jnp.int32),
            pltpu.VMEM((OUTC,), jnp.float32),
        ],
    )(verts_flat, faces_flat)
    return out.reshape(B, F, 9)


# speedup vs baseline: 129.5312x; 12.1672x over previous
import functools

import jax
import jax.numpy as jnp
from jax import lax
from jax.experimental import pallas as pl
from jax.experimental.pallas import tpu as pltpu
from jax.experimental.pallas import tpu_sc as plsc

B = 32
V = 25000
F = 50000
L = 16
NC, NS = 2, 16

CH = 1024


def _body(vT, fT, oT, verts_v, faces_v, out_v, verts_sp, faces_sp, out_sp):
    c = lax.axis_index("c")
    s = lax.axis_index("s")
    b0 = c * NS

    for h in range(8):
        @pl.when(s == 0)
        def _():
            for k in range(3):
                pltpu.sync_copy(
                    vT.at[k, pl.ds(b0 + 2 * h, 2), :], verts_sp.at[k]
                )

        plsc.subcore_barrier()

        @pl.when((s >= 2 * h) & (s < 2 * h + 2))
        def _():
            for k in range(3):
                pltpu.sync_copy(verts_sp.at[k, s - 2 * h], verts_v.at[k])

        plsc.subcore_barrier()

    def do_chunk(f0, w):
        nj = w // L

        @pl.when(s == 0)
        def _():
            for j in range(3):
                pltpu.sync_copy(
                    fT.at[j, pl.ds(b0, NS), pl.ds(f0, w)],
                    faces_sp.at[j, :, pl.ds(0, w)],
                )

        plsc.subcore_barrier()
        for j in range(3):
            pltpu.sync_copy(
                faces_sp.at[j, s, pl.ds(0, w)], faces_v.at[j, pl.ds(0, w)]
            )

        @plsc.parallel_loop(0, nj, step=1, unroll=4)
        def _loop(i):
            for j in range(3):
                fj = faces_v[j, pl.ds(i * L, L)]
                for k in range(3):
                    row = jnp.full((L,), k, jnp.int32)
                    vals = plsc.load_gather(verts_v, [row, fj])
                    out_v[3 * j + k, pl.ds(i * L, L)] = vals

        for r in range(9):
            pltpu.sync_copy(
                out_v.at[r, pl.ds(0, w)], out_sp.at[r, s, pl.ds(0, w)]
            )
        plsc.subcore_barrier()

        @pl.when(s == 0)
        def _():
            for r in range(9):
                pltpu.sync_copy(
                    out_sp.at[r, :, pl.ds(0, w)],
                    oT.at[r, pl.ds(b0, NS), pl.ds(f0, w)],
                )

    nfull = F // CH

    def chunk_step(ci, carry):
        do_chunk(pl.multiple_of(ci * CH, 128), CH)
        return carry

    lax.fori_loop(0, nfull, chunk_step, 0)
    if F % CH:
        do_chunk(nfull * CH, F % CH)


@functools.partial(jax.jit, static_argnames=())
def kernel(vertices, faces):
    vT = vertices.transpose(2, 0, 1)
    fT = faces.transpose(2, 0, 1)
    mesh = plsc.VectorSubcoreMesh(
        core_axis_name="c", subcore_axis_name="s", num_cores=NC, num_subcores=NS
    )
    outT = pl.kernel(
        _body,
        out_type=jax.ShapeDtypeStruct((9, B, F), jnp.float32),
        mesh=mesh,
        compiler_params=pltpu.CompilerParams(
            needs_layout_passes=False, use_tc_tiling_on_sc=False
        ),
        scratch_types=[
            pltpu.VMEM((3, V), jnp.float32),
            pltpu.VMEM((3, CH), jnp.int32),
            pltpu.VMEM((9, CH), jnp.float32),
            pltpu.VMEM_SHARED((3, 2, V), jnp.float32),
            pltpu.VMEM_SHARED((3, NS, CH), jnp.int32),
            pltpu.VMEM_SHARED((9, NS, CH), jnp.float32),
        ],
    )(vT, fT)
    return outT.transpose(1, 2, 0)


# speedup vs baseline: 269.1275x; 2.0777x over previous
import functools

import jax
import jax.numpy as jnp
from jax import lax
from jax.experimental import pallas as pl
from jax.experimental.pallas import tpu as pltpu
from jax.experimental.pallas import tpu_sc as plsc

B = 32
V = 25000
F = 50000
L = 16
NC, NS = 2, 16

CH = 1024


def _body(vT, fT, oT, verts_v, faces_v, out_v, verts_sp, faces_sp, out_sp):
    c = lax.axis_index("c")
    s = lax.axis_index("s")
    b0 = c * NS

    for h in range(8):
        @pl.when(s == 0)
        def _():
            for k in range(3):
                pltpu.sync_copy(
                    vT.at[k, pl.ds(b0 + 2 * h, 2), :], verts_sp.at[k]
                )

        plsc.subcore_barrier()

        @pl.when((s >= 2 * h) & (s < 2 * h + 2))
        def _():
            for k in range(3):
                pltpu.sync_copy(verts_sp.at[k, s - 2 * h], verts_v.at[k])

        plsc.subcore_barrier()

    def do_chunk(f0, w):
        nj = w // L

        for j in range(3):
            @pl.when(s == j)
            def _():
                pltpu.sync_copy(
                    fT.at[j, pl.ds(b0, NS), pl.ds(f0, w)],
                    faces_sp.at[j, :, pl.ds(0, w)],
                )

        plsc.subcore_barrier()
        pltpu.sync_copy(
            faces_sp.at[:, s, pl.ds(0, w)], faces_v.at[:, pl.ds(0, w)]
        )

        @plsc.parallel_loop(0, nj, step=1, unroll=4)
        def _loop(i):
            for j in range(3):
                fj = faces_v[j, pl.ds(i * L, L)]
                for k in range(3):
                    row = jnp.full((L,), k, jnp.int32)
                    vals = plsc.load_gather(verts_v, [row, fj])
                    out_v[3 * j + k, pl.ds(i * L, L)] = vals

        pltpu.sync_copy(
            out_v.at[:, pl.ds(0, w)], out_sp.at[:, s, pl.ds(0, w)]
        )
        plsc.subcore_barrier()

        for r in range(9):
            @pl.when(s == r)
            def _():
                pltpu.sync_copy(
                    out_sp.at[r, :, pl.ds(0, w)],
                    oT.at[r, pl.ds(b0, NS), pl.ds(f0, w)],
                )

    nfull = F // CH

    def chunk_step(ci, carry):
        do_chunk(pl.multiple_of(ci * CH, 128), CH)
        return carry

    lax.fori_loop(0, nfull, chunk_step, 0)
    if F % CH:
        do_chunk(nfull * CH, F % CH)


@functools.partial(jax.jit, static_argnames=())
def kernel(vertices, faces):
    vT = vertices.transpose(2, 0, 1)
    fT = faces.transpose(2, 0, 1)
    mesh = plsc.VectorSubcoreMesh(
        core_axis_name="c", subcore_axis_name="s", num_cores=NC, num_subcores=NS
    )
    outT = pl.kernel(
        _body,
        out_type=jax.ShapeDtypeStruct((9, B, F), jnp.float32),
        mesh=mesh,
        compiler_params=pltpu.CompilerParams(
            needs_layout_passes=False, use_tc_tiling_on_sc=False
        ),
        scratch_types=[
            pltpu.VMEM((3, V), jnp.float32),
            pltpu.VMEM((3, CH), jnp.int32),
            pltpu.VMEM((9, CH), jnp.float32),
            pltpu.VMEM_SHARED((3, 2, V), jnp.float32),
            pltpu.VMEM_SHARED((3, NS, CH), jnp.int32),
            pltpu.VMEM_SHARED((9, NS, CH), jnp.float32),
        ],
    )(vT, fT)
    return outT.transpose(1, 2, 0)


# speedup vs baseline: 293.7145x; 1.0914x over previous
import functools

import jax
import jax.numpy as jnp
from jax import lax
from jax.experimental import pallas as pl
from jax.experimental.pallas import tpu as pltpu
from jax.experimental.pallas import tpu_sc as plsc

B = 32
V = 25000
F = 50000
L = 16
NC, NS = 2, 16

CH = 1536


def _body(vT, fT, oT, verts_v, faces_v, out_v, verts_sp, faces_sp, out_sp):
    c = lax.axis_index("c")
    s = lax.axis_index("s")
    b0 = c * NS

    for h in range(8):
        @pl.when(s == 0)
        def _():
            for k in range(3):
                pltpu.sync_copy(
                    vT.at[k, pl.ds(b0 + 2 * h, 2), :], verts_sp.at[k]
                )

        plsc.subcore_barrier()

        @pl.when((s >= 2 * h) & (s < 2 * h + 2))
        def _():
            for k in range(3):
                pltpu.sync_copy(verts_sp.at[k, s - 2 * h], verts_v.at[k])

        plsc.subcore_barrier()

    def do_chunk(f0, w):
        nj = w // L

        for j in range(3):
            @pl.when(s == j)
            def _():
                pltpu.sync_copy(
                    fT.at[j, pl.ds(b0, NS), pl.ds(f0, w)],
                    faces_sp.at[j, :, pl.ds(0, w)],
                )

        plsc.subcore_barrier()
        pltpu.sync_copy(
            faces_sp.at[:, s, pl.ds(0, w)], faces_v.at[:, pl.ds(0, w)]
        )

        @plsc.parallel_loop(0, nj, step=1, unroll=4)
        def _loop(i):
            for j in range(3):
                fj = faces_v[j, pl.ds(i * L, L)]
                for k in range(3):
                    row = jnp.full((L,), k, jnp.int32)
                    vals = plsc.load_gather(verts_v, [row, fj])
                    out_v[3 * j + k, pl.ds(i * L, L)] = vals

        pltpu.sync_copy(
            out_v.at[:, pl.ds(0, w)], out_sp.at[:, s, pl.ds(0, w)]
        )
        plsc.subcore_barrier()

        for r in range(9):
            @pl.when(s == r)
            def _():
                pltpu.sync_copy(
                    out_sp.at[r, :, pl.ds(0, w)],
                    oT.at[r, pl.ds(b0, NS), pl.ds(f0, w)],
                )

    nfull = F // CH

    def chunk_step(ci, carry):
        do_chunk(pl.multiple_of(ci * CH, 128), CH)
        return carry

    lax.fori_loop(0, nfull, chunk_step, 0)
    if F % CH:
        do_chunk(nfull * CH, F % CH)


@functools.partial(jax.jit, static_argnames=())
def kernel(vertices, faces):
    vT = vertices.transpose(2, 0, 1)
    fT = faces.transpose(2, 0, 1)
    mesh = plsc.VectorSubcoreMesh(
        core_axis_name="c", subcore_axis_name="s", num_cores=NC, num_subcores=NS
    )
    outT = pl.kernel(
        _body,
        out_type=jax.ShapeDtypeStruct((9, B, F), jnp.float32),
        mesh=mesh,
        compiler_params=pltpu.CompilerParams(
            needs_layout_passes=False, use_tc_tiling_on_sc=False
        ),
        scratch_types=[
            pltpu.VMEM((3, V), jnp.float32),
            pltpu.VMEM((3, CH), jnp.int32),
            pltpu.VMEM((9, CH), jnp.float32),
            pltpu.VMEM_SHARED((3, 2, V), jnp.float32),
            pltpu.VMEM_SHARED((3, NS, CH), jnp.int32),
            pltpu.VMEM_SHARED((9, NS, CH), jnp.float32),
        ],
    )(vT, fT)
    return outT.transpose(1, 2, 0)
